# baseline (speedup 1.0000x reference)
import functools

import jax
import jax.numpy as jnp
from jax.experimental import pallas as pl
from jax.experimental.pallas import tpu as pltpu

_CT = 2048
_K = 4
_LANES = 128


def _stats_kern(nc, q_ref, w_ref, b_ref, m_ref, s_ref, m128_ref, s128_ref):
    c = pl.program_id(0)
    logits = jax.lax.dot_general(
        q_ref[:], w_ref[:], (((1,), (1,)), ((), ())),
        preferred_element_type=jnp.float32)
    logits = logits + b_ref[:]
    nk = logits.shape[1] // _LANES

    m_old = jnp.where(c == 0, jnp.float32(-1e30), m128_ref[:])
    s_old = jnp.where(c == 0, jnp.float32(0.0), s128_ref[:])
    m_new = m_old
    for k in range(nk):
        m_new = jnp.maximum(m_new, logits[:, k * _LANES:(k + 1) * _LANES])
    s_acc = jnp.zeros_like(m_new)
    for k in range(nk):
        s_acc = s_acc + jnp.exp(logits[:, k * _LANES:(k + 1) * _LANES] - m_new)
    s_new = s_old * jnp.exp(m_old - m_new) + s_acc
    m128_ref[:] = m_new
    s128_ref[:] = s_new

    @pl.when(c == nc - 1)
    def _():
        m_row = jnp.max(m_new, axis=1, keepdims=True)
        s_row = jnp.sum(s_new * jnp.exp(m_new - m_row), axis=1, keepdims=True)
        m_ref[:] = m_row
        s_ref[:] = 1.0 / s_row


def _attn_kern(nc, cap, q_ref, w_ref, b_ref, mem_ref, m_ref, s_ref,
               ret_ref, attn_ref, buf, tail_buf, sem, tail_sem):
    c = pl.program_id(0)
    tw = cap - (nc - 1) * _CT
    twa = (tw // _LANES) * _LANES
    rem = tw - twa
    logits = jax.lax.dot_general(
        q_ref[:], w_ref[:], (((1,), (1,)), ((), ())),
        preferred_element_type=jnp.float32)
    logits = logits + b_ref[:]
    e = jnp.exp(logits - m_ref[:])
    slot = jax.lax.rem(c, _K)

    @pl.when(c >= _K)
    def _():
        pltpu.make_async_copy(
            buf.at[slot],
            attn_ref.at[:, pl.ds((c - _K) * _CT, _CT)],
            sem.at[slot]).wait()

    buf[slot] = e * s_ref[:]

    @pl.when(c < nc - 1)
    def _():
        pltpu.make_async_copy(
            buf.at[slot],
            attn_ref.at[:, pl.ds(c * _CT, _CT)],
            sem.at[slot]).start()

    contrib = jax.lax.dot_general(
        e.astype(jnp.bfloat16), mem_ref[:], (((1,), (0,)), ((), ())),
        preferred_element_type=jnp.float32)

    @pl.when(c == 0)
    def _():
        ret_ref[:] = contrib

    @pl.when(c > 0)
    def _():
        ret_ref[:] = ret_ref[:] + contrib

    @pl.when(c == nc - 1)
    def _():
        ret_ref[:] = ret_ref[:] * s_ref[:]
        last = nc - 1
        pltpu.make_async_copy(
            buf.at[slot, :, pl.ds(0, twa)],
            attn_ref.at[:, pl.ds(last * _CT, twa)],
            sem.at[slot]).start()
        if rem:
            tail_buf[:] = (e * s_ref[:])[:, twa:tw]
            pltpu.make_async_copy(
                tail_buf,
                attn_ref.at[:, pl.ds(last * _CT + twa, rem)],
                tail_sem).start()
        for i in range(1, _K):
            step = last - _K + i
            pltpu.make_async_copy(
                buf.at[step % _K],
                attn_ref.at[:, pl.ds(step * _CT, _CT)],
                sem.at[step % _K]).wait()
        pltpu.make_async_copy(
            buf.at[slot, :, pl.ds(0, twa)],
            attn_ref.at[:, pl.ds(last * _CT, twa)],
            sem.at[slot]).wait()
        if rem:
            pltpu.make_async_copy(
                tail_buf,
                attn_ref.at[:, pl.ds(last * _CT + twa, rem)],
                tail_sem).wait()


def kernel(da_query, da_waaagh_memory, W_access, b_access):
    b_dim, d = da_query.shape
    cap = W_access.shape[0]
    nc = pl.cdiv(cap, _CT)
    cp = nc * _CT
    pad = cp - cap
    qb = da_query.astype(jnp.bfloat16)
    wp = jnp.pad(W_access, ((0, pad), (0, 0))).astype(jnp.bfloat16)
    memp = jnp.pad(da_waaagh_memory, ((0, pad), (0, 0))).astype(jnp.bfloat16)
    bp = jnp.pad(b_access.reshape(1, cap), ((0, 0), (0, pad)),
                 constant_values=-1e30)

    m_row, s_inv = pl.pallas_call(
        functools.partial(_stats_kern, nc),
        grid=(nc,),
        in_specs=[
            pl.BlockSpec((b_dim, d), lambda c: (0, 0)),
            pl.BlockSpec((_CT, d), lambda c: (c, 0)),
            pl.BlockSpec((1, _CT), lambda c: (0, c)),
        ],
        out_specs=[
            pl.BlockSpec((b_dim, 1), lambda c: (0, 0)),
            pl.BlockSpec((b_dim, 1), lambda c: (0, 0)),
        ],
        out_shape=[
            jax.ShapeDtypeStruct((b_dim, 1), jnp.float32),
            jax.ShapeDtypeStruct((b_dim, 1), jnp.float32),
        ],
        scratch_shapes=[
            pltpu.VMEM((b_dim, _LANES), jnp.float32),
            pltpu.VMEM((b_dim, _LANES), jnp.float32),
        ],
    )(qb, wp, bp)

    ret, attn = pl.pallas_call(
        functools.partial(_attn_kern, nc, cap),
        grid=(nc,),
        in_specs=[
            pl.BlockSpec((b_dim, d), lambda c: (0, 0)),
            pl.BlockSpec((_CT, d), lambda c: (c, 0)),
            pl.BlockSpec((1, _CT), lambda c: (0, c)),
            pl.BlockSpec((_CT, d), lambda c: (c, 0)),
            pl.BlockSpec((b_dim, 1), lambda c: (0, 0)),
            pl.BlockSpec((b_dim, 1), lambda c: (0, 0)),
        ],
        out_specs=[
            pl.BlockSpec((b_dim, d), lambda c: (0, 0)),
            pl.BlockSpec(memory_space=pl.ANY),
        ],
        out_shape=[
            jax.ShapeDtypeStruct((b_dim, d), jnp.float32),
            jax.ShapeDtypeStruct((b_dim, cap), jnp.float32),
        ],
        scratch_shapes=[
            pltpu.VMEM((_K, b_dim, _CT), jnp.float32),
            pltpu.VMEM((b_dim, cap % _LANES), jnp.float32),
            pltpu.SemaphoreType.DMA((_K,)),
            pltpu.SemaphoreType.DMA,
        ],
    )(qb, wp, bp, memp, m_row, s_inv)

    return (ret, attn)


# speedup vs baseline: 1.1910x; 1.1910x over previous
import functools

import jax
import jax.numpy as jnp
from jax.experimental import pallas as pl
from jax.experimental.pallas import tpu as pltpu

_CT = 4096
_LANES = 128


def _stats_kern(nc, q_ref, w_ref, b_ref, m_ref, s_ref, m128_ref, s128_ref):
    c = pl.program_id(0)
    logits = jax.lax.dot_general(
        q_ref[:], w_ref[:], (((1,), (1,)), ((), ())),
        preferred_element_type=jnp.float32)
    logits = logits + b_ref[:]
    nk = logits.shape[1] // _LANES

    m_old = jnp.where(c == 0, jnp.float32(-1e30), m128_ref[:])
    s_old = jnp.where(c == 0, jnp.float32(0.0), s128_ref[:])
    m_new = m_old
    for k in range(nk):
        m_new = jnp.maximum(m_new, logits[:, k * _LANES:(k + 1) * _LANES])
    s_acc = jnp.zeros_like(m_new)
    for k in range(nk):
        s_acc = s_acc + jnp.exp(logits[:, k * _LANES:(k + 1) * _LANES] - m_new)
    s_new = s_old * jnp.exp(m_old - m_new) + s_acc
    m128_ref[:] = m_new
    s128_ref[:] = s_new

    @pl.when(c == nc - 1)
    def _():
        m_row = jnp.max(m_new, axis=1, keepdims=True)
        s_row = jnp.sum(s_new * jnp.exp(m_new - m_row), axis=1, keepdims=True)
        m_ref[:] = m_row
        s_ref[:] = 1.0 / s_row


def _attn_kern(nc, q_ref, w_ref, b_ref, mem_ref, m_ref, s_ref,
               ret_ref, attn_ref):
    c = pl.program_id(0)
    logits = jax.lax.dot_general(
        q_ref[:], w_ref[:], (((1,), (1,)), ((), ())),
        preferred_element_type=jnp.float32)
    logits = logits + b_ref[:]
    e = jnp.exp(logits - m_ref[:])
    attn_ref[:] = e * s_ref[:]
    contrib = jax.lax.dot_general(
        e.astype(jnp.bfloat16), mem_ref[:], (((1,), (0,)), ((), ())),
        preferred_element_type=jnp.float32)

    @pl.when(c == 0)
    def _():
        ret_ref[:] = contrib

    @pl.when(c > 0)
    def _():
        ret_ref[:] = ret_ref[:] + contrib

    @pl.when(c == nc - 1)
    def _():
        ret_ref[:] = ret_ref[:] * s_ref[:]


def kernel(da_query, da_waaagh_memory, W_access, b_access):
    b_dim, d = da_query.shape
    cap = W_access.shape[0]
    nc = pl.cdiv(cap, _CT)
    cp = nc * _CT
    pad = cp - cap
    qb = da_query.astype(jnp.bfloat16)
    wp = jnp.pad(W_access, ((0, pad), (0, 0))).astype(jnp.bfloat16)
    memp = jnp.pad(da_waaagh_memory, ((0, pad), (0, 0))).astype(jnp.bfloat16)
    bp = jnp.pad(b_access.reshape(1, cap), ((0, 0), (0, pad)),
                 constant_values=-1e30)

    m_row, s_inv = pl.pallas_call(
        functools.partial(_stats_kern, nc),
        grid=(nc,),
        in_specs=[
            pl.BlockSpec((b_dim, d), lambda c: (0, 0)),
            pl.BlockSpec((_CT, d), lambda c: (c, 0)),
            pl.BlockSpec((1, _CT), lambda c: (0, c)),
        ],
        out_specs=[
            pl.BlockSpec((b_dim, 1), lambda c: (0, 0)),
            pl.BlockSpec((b_dim, 1), lambda c: (0, 0)),
        ],
        out_shape=[
            jax.ShapeDtypeStruct((b_dim, 1), jnp.float32),
            jax.ShapeDtypeStruct((b_dim, 1), jnp.float32),
        ],
        scratch_shapes=[
            pltpu.VMEM((b_dim, _LANES), jnp.float32),
            pltpu.VMEM((b_dim, _LANES), jnp.float32),
        ],
        compiler_params=pltpu.CompilerParams(
            vmem_limit_bytes=100 * 1024 * 1024),
    )(qb, wp, bp)

    ret, attn = pl.pallas_call(
        functools.partial(_attn_kern, nc),
        grid=(nc,),
        in_specs=[
            pl.BlockSpec((b_dim, d), lambda c: (0, 0)),
            pl.BlockSpec((_CT, d), lambda c: (c, 0)),
            pl.BlockSpec((1, _CT), lambda c: (0, c)),
            pl.BlockSpec((_CT, d), lambda c: (c, 0)),
            pl.BlockSpec((b_dim, 1), lambda c: (0, 0)),
            pl.BlockSpec((b_dim, 1), lambda c: (0, 0)),
        ],
        out_specs=[
            pl.BlockSpec((b_dim, d), lambda c: (0, 0)),
            pl.BlockSpec((b_dim, _CT), lambda c: (0, c)),
        ],
        out_shape=[
            jax.ShapeDtypeStruct((b_dim, d), jnp.float32),
            jax.ShapeDtypeStruct((b_dim, cap), jnp.float32),
        ],
        compiler_params=pltpu.CompilerParams(
            vmem_limit_bytes=100 * 1024 * 1024),
    )(qb, wp, bp, memp, m_row, s_inv)
    return (ret, attn)
